# baseline (speedup 1.0000x reference)
import functools

import jax
import jax.numpy as jnp
from jax import lax
from jax.experimental import pallas as pl
from jax.experimental.pallas import tpu as pltpu
from jax.experimental.pallas import tpu_sc as plsc

G = 256
ND = 128
ED = 16
PD = 128
N = 10000
E = 320000

NC, NS = 2, 16
NW = NC * NS

EC = 1000
E_K = 10
SC_GROUPS = [(0, 128), (128, 128), (256, 128), (384, 128),
             (512, 128), (640, 128), (768, 128), (896, 104)]
NCH = 128
N_FULL = N // NCH
N_TAIL = N - N_FULL * NCH
N_K = (N_FULL + NW - 1) // NW


def _lanesum16(x):
  idx = lax.iota(jnp.int32, 16)
  for sh in (8, 4, 2, 1):
    x = x + x.at[jnp.bitwise_xor(idx, sh)].get(mode="promise_in_bounds")
  return x


def _lane(x, i):
  return x.at[jnp.full((16,), i, jnp.int32)].get(mode="promise_in_bounds")


ACOL = 32768
APAD = 327680


def _tc_alpha_body(eft_ref, wge_ref, bge_ref, o_ref):
  x = eft_ref[...]
  a = jnp.dot(wge_ref[...], x, preferred_element_type=jnp.float32)
  o_ref[...] = (a + bge_ref[0, 0]).reshape(ACOL // 128, 128)


_tc_alpha = pl.pallas_call(
    _tc_alpha_body,
    grid=(APAD // ACOL,),
    in_specs=[
        pl.BlockSpec((ED, ACOL), lambda i: (0, i)),
        pl.BlockSpec((1, ED), lambda i: (0, 0)),
        pl.BlockSpec((1, 1), lambda i: (0, 0), memory_space=pltpu.SMEM),
    ],
    out_specs=pl.BlockSpec((ACOL // 128, 128), lambda i: (i, 0)),
    out_shape=jax.ShapeDtypeStruct((APAD // 128, 128), jnp.float32),
)


def _sc_edge_body(ef, eids, alphas,
                  epart,
                  ebuf, egbuf, eidb, abuf,
                  sem_in0, sem_in1, sem_sc0, sem_sc1,
                  epool):
  c = lax.axis_index("c")
  s = lax.axis_index("s")
  wid = s * NC + c

  @pl.when(s == 0)
  def _zero():
    zero16 = jnp.zeros((16,), jnp.float32)

    def zrow(i, carry):
      egbuf[0, i, :] = zero16
      return carry

    lax.fori_loop(0, G, zrow, 0)
    pltpu.sync_copy(egbuf.at[0, pl.ds(0, G)], epool)

  plsc.subcore_barrier()

  sem_in = [sem_in0, sem_in1]
  sem_sc = [sem_sc0, sem_sc1]
  descs_in = [None, None]
  descs_sc = [[], []]
  e0 = wid * (E_K * EC)
  r0 = e0 // 8

  def start_feat(k):
    b = k % 2
    descs_in[b] = pltpu.async_copy(
        ef.at[pl.ds(r0 + k * (EC // 8), EC // 8)], ebuf.at[b], sem_in[b])

  start_feat(0)
  for k in range(E_K):
    b = k % 2
    for d in descs_sc[b]:
      d.wait()
    descs_sc[b] = []
    pltpu.sync_copy(eids.at[pl.ds(e0 + k * EC, EC)],
                    eidb.at[b, pl.ds(0, EC)])
    pltpu.sync_copy(alphas.at[pl.ds(e0 + k * EC, EC)],
                    abuf.at[b, pl.ds(0, EC)])
    if k + 1 < E_K:
      start_feat(k + 1)
    descs_in[b].wait()

    def gate16(g, carry):
      av = abuf[b, pl.ds(g * 16, 16)]
      for h in range(2):
        j = g * 2 + h
        for u in range(8):
          egbuf[b, j * 8 + u, :] = (
              ebuf[b, j, u * 16:(u + 1) * 16] * _lane(av, h * 8 + u))
      return carry

    lax.fori_loop(0, (EC // 8) // 2, gate16, 0)
    av = abuf[b, pl.ds(EC - 8, 16)]
    j = EC // 8 - 1
    for u in range(8):
      egbuf[b, j * 8 + u, :] = (
          ebuf[b, j, u * 16:(u + 1) * 16] * _lane(av, u))

    for off, cnt in SC_GROUPS:
      descs_sc[b].append(pltpu.async_copy(
          egbuf.at[b, pl.ds(off, cnt)],
          epool.at[eidb.at[b, pl.ds(off, cnt)]], sem_sc[b], add=True))

  for b in (0, 1):
    for d in descs_sc[b]:
      d.wait()

  plsc.subcore_barrier()

  @pl.when(s == 0)
  def _writeout():
    pltpu.sync_copy(epool, epart.at[c])


_sc_edge = functools.partial(
    pl.kernel,
    out_type=jax.ShapeDtypeStruct((NC, G, ED), jnp.float32),
    mesh=plsc.VectorSubcoreMesh(core_axis_name="c", subcore_axis_name="s"),
    compiler_params=pltpu.CompilerParams(use_tc_tiling_on_sc=False),
    scratch_types=(
        pltpu.VMEM((2, EC // 8, 128), jnp.float32),
        pltpu.VMEM((2, EC, ED), jnp.float32),
        pltpu.VMEM((2, 1024), jnp.int32),
        pltpu.VMEM((2, 1024), jnp.float32),
        pltpu.SemaphoreType.DMA,
        pltpu.SemaphoreType.DMA,
        pltpu.SemaphoreType.DMA,
        pltpu.SemaphoreType.DMA,
        pltpu.VMEM_SHARED((G, ED), jnp.float32),
    ),
)(_sc_edge_body)


def _sc_node_body(nf, nids, wgn, bgn,
                  npart,
                  nbuf, ngbuf, nidb, wgnb, bgnb,
                  npool):
  c = lax.axis_index("c")
  s = lax.axis_index("s")
  wid = s * NC + c

  pltpu.sync_copy(wgn, wgnb)
  pltpu.sync_copy(bgn, bgnb)

  @pl.when(s == 0)
  def _zero():
    zero16 = jnp.zeros((16,), jnp.float32)

    def zrow(i, carry):
      for cc in range(8):
        ngbuf[i, cc * 16:(cc + 1) * 16] = zero16
      return carry

    lax.fori_loop(0, NCH, zrow, 0)
    pltpu.sync_copy(ngbuf, npool.at[pl.ds(0, NCH)])
    pltpu.sync_copy(ngbuf, npool.at[pl.ds(NCH, NCH)])

  plsc.subcore_barrier()

  wgnv = [wgnb[cc * 16:(cc + 1) * 16] for cc in range(8)]
  bgnv = bgnb[:]

  def node_chunk(base, n_nodes):
    base = pl.multiple_of(base, 8)
    pltpu.sync_copy(nf.at[pl.ds(base, n_nodes)], nbuf.at[pl.ds(0, n_nodes)])
    pltpu.sync_copy(nids.at[pl.ds(base, n_nodes)], nidb.at[pl.ds(0, n_nodes)])

    def ngate(g, carry):
      for u in range(2):
        j = g * 2 + u
        acc = jnp.zeros((16,), jnp.float32)
        rows = []
        for cc in range(8):
          rr = nbuf[j, cc * 16:(cc + 1) * 16]
          rows.append(rr)
          acc = acc + rr * wgnv[cc]
        a = _lanesum16(acc) + bgnv
        for cc in range(8):
          ngbuf[j, cc * 16:(cc + 1) * 16] = rows[cc] * a
      return carry

    lax.fori_loop(0, n_nodes // 2, ngate, 0)
    for r in range(n_nodes // 16):
      pltpu.sync_copy(ngbuf.at[pl.ds(r * 16, 16)],
                      npool.at[nidb.at[pl.ds(r * 16, 16)]], add=True)

  for k in range(N_K):
    ncid = wid + NW * k

    @pl.when(ncid < N_FULL)
    def _node_full():
      node_chunk(ncid * NCH, NCH)

  @pl.when(wid == NW - 2)
  def _ntail():
    node_chunk(N_FULL * NCH, N_TAIL)

  plsc.subcore_barrier()

  @pl.when(s == 0)
  def _writeout():
    pltpu.sync_copy(npool, npart.at[c])


_sc_node = functools.partial(
    pl.kernel,
    out_type=jax.ShapeDtypeStruct((NC, G, ND), jnp.float32),
    mesh=plsc.VectorSubcoreMesh(core_axis_name="c", subcore_axis_name="s"),
    compiler_params=pltpu.CompilerParams(use_tc_tiling_on_sc=False),
    scratch_types=(
        pltpu.VMEM((NCH, ND), jnp.float32),
        pltpu.VMEM((NCH, ND), jnp.float32),
        pltpu.VMEM((NCH,), jnp.int32),
        pltpu.VMEM((128,), jnp.float32),
        pltpu.VMEM((16,), jnp.float32),
        pltpu.VMEM_SHARED((G, ND), jnp.float32),
    ),
)(_sc_node_body)


def _tc_finish_body(np_ref, ep_ref, wpn_ref, wpe_ref, bp_ref, o_ref):
  pooled_n = np_ref[0] + np_ref[1]
  pooled_e = ep_ref[0] + ep_ref[1]
  o_ref[...] = (
      jnp.dot(pooled_n, wpn_ref[...], preferred_element_type=jnp.float32)
      + jnp.dot(pooled_e, wpe_ref[...], preferred_element_type=jnp.float32)
      + bp_ref[...])


_tc_finish = pl.pallas_call(
    _tc_finish_body,
    out_shape=jax.ShapeDtypeStruct((G, PD), jnp.float32),
)


def kernel(node_features, edge_features, node_batch_list, edge_batch_list,
           Wg_n, bg_n, Wg_e, bg_e, Wp, bp):
  nids = node_batch_list.astype(jnp.int32)
  eids = edge_batch_list.astype(jnp.int32)
  wgn = Wg_n.reshape(ND)
  bgn = jnp.full((16,), bg_n[0], jnp.float32)

  eft = edge_features.T
  alphas = _tc_alpha(eft, Wg_e.reshape(1, ED), bg_e.reshape(1, 1))
  ef8 = edge_features.reshape(E // 8, 8 * ED)

  npart = _sc_node(node_features, nids, wgn, bgn)
  epart = _sc_edge(ef8, eids, alphas.reshape(APAD))
  return _tc_finish(npart, epart, Wp[:ND], Wp[ND:], bp.reshape(1, PD))


# speedup vs baseline: 1.0806x; 1.0806x over previous
import functools

import jax
import jax.numpy as jnp
from jax import lax
from jax.experimental import pallas as pl
from jax.experimental.pallas import tpu as pltpu
from jax.experimental.pallas import tpu_sc as plsc

G = 256
ND = 128
ED = 16
PD = 128
N = 10000
E = 320000

NC, NS = 2, 16
NW = NC * NS

EC = 1000
E_K = 10
SC_GROUPS = [(0, 128), (128, 128), (256, 128), (384, 128),
             (512, 128), (640, 128), (768, 128), (896, 104)]
NCH = 128
N_FULL = N // NCH
N_TAIL = N - N_FULL * NCH
N_K = (N_FULL + NW - 1) // NW


def _lanesum16(x):
  idx = lax.iota(jnp.int32, 16)
  for sh in (8, 4, 2, 1):
    x = x + x.at[jnp.bitwise_xor(idx, sh)].get(mode="promise_in_bounds")
  return x


def _lane(x, i):
  return x.at[jnp.full((16,), i, jnp.int32)].get(mode="promise_in_bounds")


ACOL = 32768
APAD = 327680


def _tc_alpha_body(eft_ref, wge_ref, bge_ref, o_ref):
  x = eft_ref[...]
  a = jnp.dot(wge_ref[...], x, preferred_element_type=jnp.float32)
  o_ref[...] = (a + bge_ref[0, 0]).reshape(ACOL // 128, 128)


_tc_alpha = pl.pallas_call(
    _tc_alpha_body,
    grid=(APAD // ACOL,),
    in_specs=[
        pl.BlockSpec((ED, ACOL), lambda i: (0, i)),
        pl.BlockSpec((1, ED), lambda i: (0, 0)),
        pl.BlockSpec((1, 1), lambda i: (0, 0), memory_space=pltpu.SMEM),
    ],
    out_specs=pl.BlockSpec((ACOL // 128, 128), lambda i: (i, 0)),
    out_shape=jax.ShapeDtypeStruct((APAD // 128, 128), jnp.float32),
)


def _sc_edge_body(ef, eids, alphas, dep,
                  epart,
                  ebuf, egbuf, eidb, abuf,
                  sem_in0, sem_in1, sem_sc0, sem_sc1,
                  epool):
  c = lax.axis_index("c")
  s = lax.axis_index("s")
  wid = s * NC + c

  @pl.when(s == 0)
  def _zero():
    zero16 = jnp.zeros((16,), jnp.float32)

    def zrow(i, carry):
      egbuf[0, i, :] = zero16
      return carry

    lax.fori_loop(0, G, zrow, 0)
    pltpu.sync_copy(egbuf.at[0, pl.ds(0, G)], epool)

  plsc.subcore_barrier()

  sem_in = [sem_in0, sem_in1]
  sem_sc = [sem_sc0, sem_sc1]
  descs_in = [None, None]
  descs_sc = [[], []]
  e0 = wid * (E_K * EC)
  r0 = e0 // 8

  def start_feat(k):
    b = k % 2
    descs_in[b] = pltpu.async_copy(
        ef.at[pl.ds(r0 + k * (EC // 8), EC // 8)], ebuf.at[b], sem_in[b])

  start_feat(0)
  for k in range(E_K):
    b = k % 2
    for d in descs_sc[b]:
      d.wait()
    descs_sc[b] = []
    pltpu.sync_copy(eids.at[pl.ds(e0 + k * EC, EC)],
                    eidb.at[b, pl.ds(0, EC)])
    pltpu.sync_copy(alphas.at[pl.ds(e0 + k * EC, EC)],
                    abuf.at[b, pl.ds(0, EC)])
    if k + 1 < E_K:
      start_feat(k + 1)
    descs_in[b].wait()

    @plsc.parallel_loop(0, (EC // 8) // 2, step=1, unroll=2)
    def gate16(g):
      av = abuf[b, pl.ds(g * 16, 16)]
      for h in range(2):
        j = g * 2 + h
        for u in range(8):
          egbuf[b, j * 8 + u, :] = (
              ebuf[b, j, u * 16:(u + 1) * 16] * _lane(av, h * 8 + u))
    av = abuf[b, pl.ds(EC - 8, 16)]
    j = EC // 8 - 1
    for u in range(8):
      egbuf[b, j * 8 + u, :] = (
          ebuf[b, j, u * 16:(u + 1) * 16] * _lane(av, u))

    for off, cnt in SC_GROUPS:
      descs_sc[b].append(pltpu.async_copy(
          egbuf.at[b, pl.ds(off, cnt)],
          epool.at[eidb.at[b, pl.ds(off, cnt)]], sem_sc[b], add=True))

  for b in (0, 1):
    for d in descs_sc[b]:
      d.wait()

  plsc.subcore_barrier()

  @pl.when(s == 0)
  def _writeout():
    pltpu.sync_copy(epool, epart.at[c])


_sc_edge = functools.partial(
    pl.kernel,
    out_type=jax.ShapeDtypeStruct((NC, G, ED), jnp.float32),
    mesh=plsc.VectorSubcoreMesh(core_axis_name="c", subcore_axis_name="s"),
    compiler_params=pltpu.CompilerParams(use_tc_tiling_on_sc=False),
    scratch_types=(
        pltpu.VMEM((2, EC // 8, 128), jnp.float32),
        pltpu.VMEM((2, EC, ED), jnp.float32),
        pltpu.VMEM((2, 1024), jnp.int32),
        pltpu.VMEM((2, 1024), jnp.float32),
        pltpu.SemaphoreType.DMA,
        pltpu.SemaphoreType.DMA,
        pltpu.SemaphoreType.DMA,
        pltpu.SemaphoreType.DMA,
        pltpu.VMEM_SHARED((G, ED), jnp.float32),
    ),
)(_sc_edge_body)


def _sc_node_body(nf, nids, wgn, bgn,
                  npart,
                  nbuf, ngbuf, nidb, wgnb, bgnb,
                  npool):
  c = lax.axis_index("c")
  s = lax.axis_index("s")
  wid = s * NC + c

  pltpu.sync_copy(wgn, wgnb)
  pltpu.sync_copy(bgn, bgnb)

  @pl.when(s == 0)
  def _zero():
    zero16 = jnp.zeros((16,), jnp.float32)

    def zrow(i, carry):
      for cc in range(8):
        ngbuf[i, cc * 16:(cc + 1) * 16] = zero16
      return carry

    lax.fori_loop(0, NCH, zrow, 0)
    pltpu.sync_copy(ngbuf, npool.at[pl.ds(0, NCH)])
    pltpu.sync_copy(ngbuf, npool.at[pl.ds(NCH, NCH)])

  plsc.subcore_barrier()

  wgnv = [wgnb[cc * 16:(cc + 1) * 16] for cc in range(8)]
  bgnv = bgnb[:]

  def node_chunk(base, n_nodes):
    base = pl.multiple_of(base, 8)
    pltpu.sync_copy(nf.at[pl.ds(base, n_nodes)], nbuf.at[pl.ds(0, n_nodes)])
    pltpu.sync_copy(nids.at[pl.ds(base, n_nodes)], nidb.at[pl.ds(0, n_nodes)])

    def ngate(g, carry):
      for u in range(2):
        j = g * 2 + u
        acc = jnp.zeros((16,), jnp.float32)
        rows = []
        for cc in range(8):
          rr = nbuf[j, cc * 16:(cc + 1) * 16]
          rows.append(rr)
          acc = acc + rr * wgnv[cc]
        a = _lanesum16(acc) + bgnv
        for cc in range(8):
          ngbuf[j, cc * 16:(cc + 1) * 16] = rows[cc] * a
      return carry

    lax.fori_loop(0, n_nodes // 2, ngate, 0)
    for r in range(n_nodes // 16):
      pltpu.sync_copy(ngbuf.at[pl.ds(r * 16, 16)],
                      npool.at[nidb.at[pl.ds(r * 16, 16)]], add=True)

  for k in range(N_K):
    ncid = wid + NW * k

    @pl.when(ncid < N_FULL)
    def _node_full():
      node_chunk(ncid * NCH, NCH)

  @pl.when(wid == NW - 2)
  def _ntail():
    node_chunk(N_FULL * NCH, N_TAIL)

  plsc.subcore_barrier()

  @pl.when(s == 0)
  def _writeout():
    pltpu.sync_copy(npool, npart.at[c])


_sc_node = functools.partial(
    pl.kernel,
    out_type=jax.ShapeDtypeStruct((NC, G, ND), jnp.float32),
    mesh=plsc.VectorSubcoreMesh(core_axis_name="c", subcore_axis_name="s"),
    compiler_params=pltpu.CompilerParams(use_tc_tiling_on_sc=False),
    scratch_types=(
        pltpu.VMEM((NCH, ND), jnp.float32),
        pltpu.VMEM((NCH, ND), jnp.float32),
        pltpu.VMEM((NCH,), jnp.int32),
        pltpu.VMEM((128,), jnp.float32),
        pltpu.VMEM((16,), jnp.float32),
        pltpu.VMEM_SHARED((G, ND), jnp.float32),
    ),
)(_sc_node_body)


def _tc_finish_body(np_ref, ep_ref, wpn_ref, wpe_ref, bp_ref, o_ref):
  pooled_n = np_ref[0] + np_ref[1]
  pooled_e = ep_ref[0] + ep_ref[1]
  o_ref[...] = (
      jnp.dot(pooled_n, wpn_ref[...], preferred_element_type=jnp.float32)
      + jnp.dot(pooled_e, wpe_ref[...], preferred_element_type=jnp.float32)
      + bp_ref[...])


_tc_finish = pl.pallas_call(
    _tc_finish_body,
    out_shape=jax.ShapeDtypeStruct((G, PD), jnp.float32),
)


def kernel(node_features, edge_features, node_batch_list, edge_batch_list,
           Wg_n, bg_n, Wg_e, bg_e, Wp, bp):
  nids = node_batch_list.astype(jnp.int32)
  eids = edge_batch_list.astype(jnp.int32)
  wgn = Wg_n.reshape(ND)
  bgn = jnp.full((16,), bg_n[0], jnp.float32)

  eft = edge_features.T
  alphas = _tc_alpha(eft, Wg_e.reshape(1, ED), bg_e.reshape(1, 1))
  ef8 = edge_features.reshape(E // 8, 8 * ED)

  npart = _sc_node(node_features, nids, wgn, bgn)
  epart = _sc_edge(ef8, eids, alphas.reshape(APAD), npart)
  return _tc_finish(npart, epart, Wp[:ND], Wp[ND:], bp.reshape(1, PD))
